# baseline (speedup 1.0000x reference)
import functools

import jax
import jax.numpy as jnp
from jax import lax
from jax.experimental import pallas as pl
from jax.experimental.pallas import tpu as pltpu
from jax.experimental.pallas import tpu_sc as plsc

N_NODES = 10000
D = 128
DH = D // 2
N_EDGES = 320000
N_GRAPHS = 64

NC = 2
NS = 16
CH = 128
NCHUNK = N_EDGES // CH
CPW = NCHUNK // NS
NTAIL = NCHUNK - CPW * NS
ROWS = 10240
RPT = ROWS // NS
NBUF = 12

_mesh = plsc.VectorSubcoreMesh(core_axis_name="c", subcore_axis_name="s")


@functools.partial(
    pl.kernel,
    mesh=_mesh,
    compiler_params=pltpu.CompilerParams(use_tc_tiling_on_sc=False),
    out_type=jax.ShapeDtypeStruct((NC, ROWS, DH), jnp.bfloat16),
    scratch_types=[
        pltpu.VMEM((CPW + 1, CH), jnp.int32),
        pltpu.VMEM((CPW + 1, CH), jnp.int32),
        pltpu.VMEM((NBUF, CH, DH), jnp.bfloat16),
        pltpu.VMEM_SHARED((ROWS, DH), jnp.bfloat16),
    ] + [pltpu.SemaphoreType.DMA] * NBUF,
)
def _sc_agg(hl_hbm, hr_hbm, src_hbm, dst_hbm, zeros_hbm, out_hbm,
            src_v, dst_v, rows_v, acc, *gsems):
    cid = lax.axis_index("c")
    sid = lax.axis_index("s")

    c_src = pltpu.async_copy(src_hbm.at[pl.ds(sid * CPW, CPW)],
                             src_v.at[pl.ds(0, CPW)], gsems[0])
    c_dst = pltpu.async_copy(dst_hbm.at[pl.ds(sid * CPW, CPW)],
                             dst_v.at[pl.ds(0, CPW)], gsems[1])
    c_zero = pltpu.async_copy(zeros_hbm, acc.at[pl.ds(sid * RPT, RPT)],
                              gsems[2])
    c_src.wait()
    c_dst.wait()
    c_zero.wait()

    @pl.when(sid < NTAIL)
    def _():
        pltpu.sync_copy(src_hbm.at[pl.ds(NS * CPW + sid, 1)],
                        src_v.at[pl.ds(CPW, 1)])
        pltpu.sync_copy(dst_hbm.at[pl.ds(NS * CPW + sid, 1)],
                        dst_v.at[pl.ds(CPW, 1)])

    plsc.subcore_barrier()

    def pipeline(h_hbm):
        def gcopy(c, b):
            return pltpu.make_async_copy(
                h_hbm.at[src_v.at[c]], rows_v.at[b], gsems[b])

        for b in range(NBUF - 1):
            gcopy(b, b).start()

        def body(k, carry):
            base = k * NBUF
            for b in range(NBUF):
                c = base + b
                nxt = c + NBUF - 1

                @pl.when(nxt < CPW)
                def _():
                    gcopy(nxt, (b + NBUF - 1) % NBUF).start()

                gcopy(c, b).wait()
                pltpu.sync_copy(rows_v.at[b], acc.at[dst_v.at[c]], add=True)
            return carry

        lax.fori_loop(0, CPW // NBUF, body, 0)

        @pl.when(sid < NTAIL)
        def _():
            gcopy(CPW, 0).start()
            gcopy(CPW, 0).wait()
            pltpu.sync_copy(rows_v.at[0], acc.at[dst_v.at[CPW]], add=True)

    @pl.when(cid == 0)
    def _():
        pipeline(hl_hbm)

    @pl.when(cid == 1)
    def _():
        pipeline(hr_hbm)

    plsc.subcore_barrier()

    pltpu.sync_copy(acc.at[pl.ds(sid * RPT, RPT)],
                    out_hbm.at[cid, pl.ds(sid * RPT, RPT)])


_BLK = 5000
_GRID = N_NODES // _BLK


def _layer_tc(hl, hr, p, W1, W2):
    def body(hl_ref, hr_ref, p_ref, w1_ref, w2_ref, ol_ref, or_ref):
        h = jnp.concatenate([hl_ref[...], hr_ref[...]], axis=1)
        agg = jnp.concatenate([p_ref[0], p_ref[1]], axis=1).astype(jnp.float32)
        res = jnp.maximum(
            jnp.dot(h, w1_ref[...].astype(jnp.bfloat16),
                    preferred_element_type=jnp.float32)
            + jnp.dot(agg, w2_ref[...], preferred_element_type=jnp.float32),
            0.0)
        ol_ref[...] = res[:, :DH].astype(jnp.bfloat16)
        or_ref[...] = res[:, DH:].astype(jnp.bfloat16)

    return pl.pallas_call(
        body,
        grid=(_GRID,),
        in_specs=[
            pl.BlockSpec((_BLK, DH), lambda i: (i, 0)),
            pl.BlockSpec((_BLK, DH), lambda i: (i, 0)),
            pl.BlockSpec((NC, _BLK, DH), lambda i: (0, i, 0)),
            pl.BlockSpec((D, D), lambda i: (0, 0)),
            pl.BlockSpec((D, D), lambda i: (0, 0)),
        ],
        out_specs=[
            pl.BlockSpec((_BLK, DH), lambda i: (i, 0)),
            pl.BlockSpec((_BLK, DH), lambda i: (i, 0)),
        ],
        out_shape=[
            jax.ShapeDtypeStruct((N_NODES, DH), jnp.bfloat16),
            jax.ShapeDtypeStruct((N_NODES, DH), jnp.bfloat16),
        ],
    )(hl, hr, p, W1, W2)


def _final_tc(hl, hr, p, W1, W2, batch3, Wc1, bc1, Wc2, bc2):
    def body(hl_ref, hr_ref, p_ref, w1_ref, w2_ref, b_ref, wc1_ref, bc1_ref,
             wc2_ref, bc2_ref, o_ref, pooled):
        i = pl.program_id(0)
        h = jnp.concatenate([hl_ref[...], hr_ref[...]], axis=1)
        agg = jnp.concatenate([p_ref[0], p_ref[1]], axis=1).astype(jnp.float32)
        h3 = jnp.maximum(
            jnp.dot(h, w1_ref[...].astype(jnp.bfloat16),
                    preferred_element_type=jnp.float32)
            + jnp.dot(agg, w2_ref[...], preferred_element_type=jnp.float32),
            0.0)
        bids = b_ref[0, 0, :]
        onehot_t = (lax.broadcasted_iota(jnp.int32, (N_GRAPHS, _BLK), 0)
                    == bids[None, :]).astype(jnp.float32)
        part = jnp.dot(onehot_t, h3, preferred_element_type=jnp.float32)

        @pl.when(i == 0)
        def _():
            pooled[...] = part

        @pl.when(i > 0)
        def _():
            pooled[...] += part

        @pl.when(i == _GRID - 1)
        def _():
            hid = jnp.maximum(
                jnp.dot(pooled[...], wc1_ref[...],
                        preferred_element_type=jnp.float32) + bc1_ref[...],
                0.0)
            o_ref[...] = jnp.dot(
                hid, wc2_ref[...],
                preferred_element_type=jnp.float32) + bc2_ref[...]

    return pl.pallas_call(
        body,
        grid=(_GRID,),
        in_specs=[
            pl.BlockSpec((_BLK, DH), lambda i: (i, 0)),
            pl.BlockSpec((_BLK, DH), lambda i: (i, 0)),
            pl.BlockSpec((NC, _BLK, DH), lambda i: (0, i, 0)),
            pl.BlockSpec((D, D), lambda i: (0, 0)),
            pl.BlockSpec((D, D), lambda i: (0, 0)),
            pl.BlockSpec((1, 1, _BLK), lambda i: (i, 0, 0)),
            pl.BlockSpec((D, D), lambda i: (0, 0)),
            pl.BlockSpec((1, D), lambda i: (0, 0)),
            pl.BlockSpec((D, D), lambda i: (0, 0)),
            pl.BlockSpec((1, D), lambda i: (0, 0)),
        ],
        out_specs=pl.BlockSpec((N_GRAPHS, D), lambda i: (0, 0)),
        out_shape=jax.ShapeDtypeStruct((N_GRAPHS, D), jnp.float32),
        scratch_shapes=[pltpu.VMEM((N_GRAPHS, D), jnp.float32)],
    )(hl, hr, p, W1, W2, batch3, Wc1, bc1, Wc2, bc2)


def kernel(x, edge_index, batch, W1_0, W2_0, W1_1, W2_1, W1_2, W2_2,
           Wc1, bc1, Wc2, bc2):
    src = edge_index[0]
    dst = edge_index[1]
    srcp = src.reshape(NCHUNK, CH)
    dstp = dst.reshape(NCHUNK, CH)
    zeros = jnp.zeros((RPT, DH), jnp.bfloat16)
    batch3 = batch.reshape(_GRID, 1, _BLK)
    bc1r = bc1.reshape(1, D)
    bc2r = bc2.reshape(1, D)
    hl = x[:, :DH].astype(jnp.bfloat16)
    hr = x[:, DH:].astype(jnp.bfloat16)

    p = _sc_agg(hl, hr, srcp, dstp, zeros)
    hl, hr = _layer_tc(hl, hr, p, W1_0, W2_0)
    p = _sc_agg(hl, hr, srcp, dstp, zeros)
    hl, hr = _layer_tc(hl, hr, p, W1_1, W2_1)
    p = _sc_agg(hl, hr, srcp, dstp, zeros)
    return _final_tc(hl, hr, p, W1_2, W2_2, batch3, Wc1, bc1r, Wc2, bc2r)


# speedup vs baseline: 1.0393x; 1.0393x over previous
import functools

import jax
import jax.numpy as jnp
from jax import lax
from jax.experimental import pallas as pl
from jax.experimental.pallas import tpu as pltpu
from jax.experimental.pallas import tpu_sc as plsc

N_NODES = 10000
D = 128
DH = D // 2
N_EDGES = 320000
N_GRAPHS = 64

NC = 2
NS = 16
CH = 128
NCHUNK = N_EDGES // CH
CPW = NCHUNK // NS
NTAIL = NCHUNK - CPW * NS
ROWS = 10240
RPT = ROWS // NS
NBUF = 6

_mesh = plsc.VectorSubcoreMesh(core_axis_name="c", subcore_axis_name="s")


@functools.partial(
    pl.kernel,
    mesh=_mesh,
    compiler_params=pltpu.CompilerParams(use_tc_tiling_on_sc=False),
    out_type=jax.ShapeDtypeStruct((NC, ROWS, DH), jnp.bfloat16),
    scratch_types=[
        pltpu.VMEM((CPW + 1, CH), jnp.int32),
        pltpu.VMEM((CPW + 1, CH), jnp.int32),
        pltpu.VMEM((NBUF, CH, DH), jnp.bfloat16),
        pltpu.VMEM_SHARED((ROWS, DH), jnp.bfloat16),
    ] + [pltpu.SemaphoreType.DMA] * NBUF,
)
def _sc_agg(hl_hbm, hr_hbm, src_hbm, dst_hbm, zeros_hbm, out_hbm,
            src_v, dst_v, rows_v, acc, *gsems):
    cid = lax.axis_index("c")
    sid = lax.axis_index("s")

    c_src = pltpu.async_copy(src_hbm.at[pl.ds(sid * CPW, CPW)],
                             src_v.at[pl.ds(0, CPW)], gsems[0])
    c_dst = pltpu.async_copy(dst_hbm.at[pl.ds(sid * CPW, CPW)],
                             dst_v.at[pl.ds(0, CPW)], gsems[1])
    c_zero = pltpu.async_copy(zeros_hbm, acc.at[pl.ds(sid * RPT, RPT)],
                              gsems[2])
    c_src.wait()
    c_dst.wait()
    c_zero.wait()

    @pl.when(sid < NTAIL)
    def _():
        pltpu.sync_copy(src_hbm.at[pl.ds(NS * CPW + sid, 1)],
                        src_v.at[pl.ds(CPW, 1)])
        pltpu.sync_copy(dst_hbm.at[pl.ds(NS * CPW + sid, 1)],
                        dst_v.at[pl.ds(CPW, 1)])

    plsc.subcore_barrier()

    def pipeline(h_hbm):
        def gcopy(c, b):
            return pltpu.make_async_copy(
                h_hbm.at[src_v.at[c]], rows_v.at[b], gsems[b])

        for b in range(NBUF - 1):
            gcopy(b, b).start()

        def body(k, carry):
            base = k * NBUF
            for b in range(NBUF):
                c = base + b
                nxt = c + NBUF - 1

                @pl.when(nxt < CPW)
                def _():
                    gcopy(nxt, (b + NBUF - 1) % NBUF).start()

                gcopy(c, b).wait()
                pltpu.sync_copy(rows_v.at[b], acc.at[dst_v.at[c]], add=True)
            return carry

        lax.fori_loop(0, CPW // NBUF, body, 0)

        @pl.when(sid < NTAIL)
        def _():
            gcopy(CPW, 0).start()
            gcopy(CPW, 0).wait()
            pltpu.sync_copy(rows_v.at[0], acc.at[dst_v.at[CPW]], add=True)

    @pl.when(cid == 0)
    def _():
        pipeline(hl_hbm)

    @pl.when(cid == 1)
    def _():
        pipeline(hr_hbm)

    plsc.subcore_barrier()

    pltpu.sync_copy(acc.at[pl.ds(sid * RPT, RPT)],
                    out_hbm.at[cid, pl.ds(sid * RPT, RPT)])


_BLK = 5000
_GRID = N_NODES // _BLK


def _layer_tc(hl, hr, p, W1, W2):
    def body(hl_ref, hr_ref, p_ref, w1_ref, w2_ref, ol_ref, or_ref):
        h = jnp.concatenate([hl_ref[...], hr_ref[...]], axis=1)
        agg = jnp.concatenate([p_ref[0], p_ref[1]], axis=1).astype(jnp.float32)
        res = jnp.maximum(
            jnp.dot(h, w1_ref[...].astype(jnp.bfloat16),
                    preferred_element_type=jnp.float32)
            + jnp.dot(agg, w2_ref[...], preferred_element_type=jnp.float32),
            0.0)
        ol_ref[...] = res[:, :DH].astype(jnp.bfloat16)
        or_ref[...] = res[:, DH:].astype(jnp.bfloat16)

    return pl.pallas_call(
        body,
        grid=(_GRID,),
        in_specs=[
            pl.BlockSpec((_BLK, DH), lambda i: (i, 0)),
            pl.BlockSpec((_BLK, DH), lambda i: (i, 0)),
            pl.BlockSpec((NC, _BLK, DH), lambda i: (0, i, 0)),
            pl.BlockSpec((D, D), lambda i: (0, 0)),
            pl.BlockSpec((D, D), lambda i: (0, 0)),
        ],
        out_specs=[
            pl.BlockSpec((_BLK, DH), lambda i: (i, 0)),
            pl.BlockSpec((_BLK, DH), lambda i: (i, 0)),
        ],
        out_shape=[
            jax.ShapeDtypeStruct((N_NODES, DH), jnp.bfloat16),
            jax.ShapeDtypeStruct((N_NODES, DH), jnp.bfloat16),
        ],
    )(hl, hr, p, W1, W2)


def _final_tc(hl, hr, p, W1, W2, batch3, Wc1, bc1, Wc2, bc2):
    def body(hl_ref, hr_ref, p_ref, w1_ref, w2_ref, b_ref, wc1_ref, bc1_ref,
             wc2_ref, bc2_ref, o_ref, pooled):
        i = pl.program_id(0)
        h = jnp.concatenate([hl_ref[...], hr_ref[...]], axis=1)
        agg = jnp.concatenate([p_ref[0], p_ref[1]], axis=1).astype(jnp.float32)
        h3 = jnp.maximum(
            jnp.dot(h, w1_ref[...].astype(jnp.bfloat16),
                    preferred_element_type=jnp.float32)
            + jnp.dot(agg, w2_ref[...], preferred_element_type=jnp.float32),
            0.0)
        bids = b_ref[0, 0, :]
        onehot_t = (lax.broadcasted_iota(jnp.int32, (N_GRAPHS, _BLK), 0)
                    == bids[None, :]).astype(jnp.float32)
        part = jnp.dot(onehot_t, h3, preferred_element_type=jnp.float32)

        @pl.when(i == 0)
        def _():
            pooled[...] = part

        @pl.when(i > 0)
        def _():
            pooled[...] += part

        @pl.when(i == _GRID - 1)
        def _():
            hid = jnp.maximum(
                jnp.dot(pooled[...], wc1_ref[...],
                        preferred_element_type=jnp.float32) + bc1_ref[...],
                0.0)
            o_ref[...] = jnp.dot(
                hid, wc2_ref[...],
                preferred_element_type=jnp.float32) + bc2_ref[...]

    return pl.pallas_call(
        body,
        grid=(_GRID,),
        in_specs=[
            pl.BlockSpec((_BLK, DH), lambda i: (i, 0)),
            pl.BlockSpec((_BLK, DH), lambda i: (i, 0)),
            pl.BlockSpec((NC, _BLK, DH), lambda i: (0, i, 0)),
            pl.BlockSpec((D, D), lambda i: (0, 0)),
            pl.BlockSpec((D, D), lambda i: (0, 0)),
            pl.BlockSpec((1, 1, _BLK), lambda i: (i, 0, 0)),
            pl.BlockSpec((D, D), lambda i: (0, 0)),
            pl.BlockSpec((1, D), lambda i: (0, 0)),
            pl.BlockSpec((D, D), lambda i: (0, 0)),
            pl.BlockSpec((1, D), lambda i: (0, 0)),
        ],
        out_specs=pl.BlockSpec((N_GRAPHS, D), lambda i: (0, 0)),
        out_shape=jax.ShapeDtypeStruct((N_GRAPHS, D), jnp.float32),
        scratch_shapes=[pltpu.VMEM((N_GRAPHS, D), jnp.float32)],
    )(hl, hr, p, W1, W2, batch3, Wc1, bc1, Wc2, bc2)


def kernel(x, edge_index, batch, W1_0, W2_0, W1_1, W2_1, W1_2, W2_2,
           Wc1, bc1, Wc2, bc2):
    src = edge_index[0]
    dst = edge_index[1]
    srcp = src.reshape(NCHUNK, CH)
    dstp = dst.reshape(NCHUNK, CH)
    zeros = jnp.zeros((RPT, DH), jnp.bfloat16)
    batch3 = batch.reshape(_GRID, 1, _BLK)
    bc1r = bc1.reshape(1, D)
    bc2r = bc2.reshape(1, D)
    hl = x[:, :DH].astype(jnp.bfloat16)
    hr = x[:, DH:].astype(jnp.bfloat16)

    p = _sc_agg(hl, hr, srcp, dstp, zeros)
    hl, hr = _layer_tc(hl, hr, p, W1_0, W2_0)
    p = _sc_agg(hl, hr, srcp, dstp, zeros)
    hl, hr = _layer_tc(hl, hr, p, W1_1, W2_1)
    p = _sc_agg(hl, hr, srcp, dstp, zeros)
    return _final_tc(hl, hr, p, W1_2, W2_2, batch3, Wc1, bc1r, Wc2, bc2r)
